# baseline (speedup 1.0000x reference)
import jax
import jax.numpy as jnp
from jax.experimental import pallas as pl
from jax.experimental.pallas import tpu as pltpu

_B = 4
_LATENT = 128
_N0 = 512
_CHANNELS = [32, 16, 8]
_NODES = [1024, 2048, 4096]
_CIN = [32, 32, 16]
_COUT = [32, 16, 8]
_TM = 256
_TFC = 2048
_NBUF = 5
_PREF = 3
_SUB_ROWS = 32

_DOT_KW = dict(preferred_element_type=jnp.float32,
               precision=jax.lax.Precision.DEFAULT)


def _split(x_f32):
    x_hi = x_f32.astype(jnp.bfloat16).astype(jnp.float32)
    return x_hi, x_f32 - x_hi


def _block_diag(w, c_in, c_out):
    wt = jnp.tile(w, (_B, _B))
    rows = jax.lax.broadcasted_iota(jnp.int32, wt.shape, 0) // c_in
    cols = jax.lax.broadcasted_iota(jnp.int32, wt.shape, 1) // c_out
    return jnp.where(rows == cols, wt, 0.0)


def _mega_body(lat_ref, fcb_ref, ub0_ref, w0_ref, b0_ref, ub1_ref, w1_ref,
               b1_ref, ub2_ref, w2_ref, b2_ref,
               fcw_hbm, u0_hbm, l0_hbm, u1_hbm, l1_hbm, u2_hbm, l2_hbm,
               out_ref, buf_b, buf_c, buf_d, buf_e,
               fc_out, y0, x1, y1, x2, y2, sems):
    chain = (None, fc_out, y0, x1, y1, x2, y2, out_ref)
    hbm_refs = (fcw_hbm, u0_hbm, l0_hbm, u1_hbm, l1_hbm, u2_hbm, l2_hbm)
    bufs = (buf_d, buf_b, buf_c, buf_c, buf_d, buf_d, buf_e)
    n_tiles = (_N0 * _CHANNELS[0] // _TFC,) + tuple(
        _NODES[(p - 1) // 2] // _TM for p in range(1, 7))
    ub_refs = (ub0_ref, ub1_ref, ub2_ref)
    w_refs = (w0_ref, w1_ref, w2_ref)
    b_refs = (b0_ref, b1_ref, b2_ref)

    split_cache = {}
    const_cache = {}

    def get_cat(p):
        if p not in split_cache:
            if p == 1:
                f = fc_out[...].reshape(_B, _N0, _CHANNELS[0])
                x = f.transpose(1, 0, 2).reshape(_N0, _B * _CHANNELS[0])
                x = x + jnp.tile(fcb_ref[...], (1, _B))
            else:
                x = chain[p][...]
            x_hi, x_lo = _split(x)
            split_cache[p] = jnp.concatenate([x_hi, x_lo], axis=1)
        return split_cache[p]

    def get_lat_cat():
        if 'lat' not in split_cache:
            lat_hi, lat_lo = _split(lat_ref[...])
            split_cache['lat'] = jnp.concatenate([lat_hi, lat_lo], axis=0)
        return split_cache['lat']

    def get_consts(lvl):
        if lvl not in const_cache:
            const_cache[lvl] = (
                _block_diag(w_refs[lvl][...], _CIN[lvl], _COUT[lvl]),
                jnp.tile(b_refs[lvl][...], (1, _B)))
        return const_cache[lvl]

    def fc_compute(buf, slot, i):
        def main():
            w = buf[slot, 0:_LATENT, :]
            return jnp.dot(get_lat_cat(), w, **_DOT_KW)

        def epi(ff):
            f = ff[:_B, :] + ff[_B:, :]
            fc_out[:, i * _TFC:(i + 1) * _TFC] = f
        return main, epi

    def u_compute(p, buf, slot, i):
        lvl = (p - 1) // 2

        def main():
            return jnp.dot(buf[slot], get_cat(p), **_DOT_KW)

        def epi(yy):
            n = yy.shape[1] // 2
            y = yy[:, :n] + yy[:, n:]
            ub = ub_refs[lvl][i * _TM:(i + 1) * _TM, :]
            chain[p + 1][i * _TM:(i + 1) * _TM, :] = y + jnp.tile(ub, (1, _B))
        return main, epi

    def l_compute(p, buf, slot, i):
        lvl = (p - 1) // 2

        def main():
            return jnp.dot(buf[slot], get_cat(p), **_DOT_KW)

        def epi(zz):
            n = zz.shape[1] // 2
            z = zz[:, :n] + zz[:, n:]
            wbd, bt = get_consts(lvl)
            h = jnp.dot(z, wbd, **_DOT_KW)
            r = jnp.maximum(h + bt, 0.0)
            if p < 6:
                chain[p + 1][i * _TM:(i + 1) * _TM, :] = r
            else:
                for b in range(_B):
                    out_ref[b, i * _TM:(i + 1) * _TM, :] = (
                        r[:, b * _COUT[2]:(b + 1) * _COUT[2]])
        return main, epi

    copies = []
    computes = []
    phase_of = []
    g = 0
    for p in range(7):
        for i in range(n_tiles[p]):
            slot = g % _NBUF
            subs = []
            if p == 0:
                for r in range(0, _LATENT, _SUB_ROWS):
                    subs.append(pltpu.make_async_copy(
                        hbm_refs[0].at[r:r + _SUB_ROWS,
                                       i * _TFC:(i + 1) * _TFC],
                        bufs[0].at[slot, r:r + _SUB_ROWS, :],
                        sems.at[slot]))
                computes.append(fc_compute(bufs[0], slot, i))
            else:
                for r in range(0, _TM, _SUB_ROWS):
                    subs.append(pltpu.make_async_copy(
                        hbm_refs[p].at[i * _TM + r:i * _TM + r + _SUB_ROWS, :],
                        bufs[p].at[slot, r:r + _SUB_ROWS, :], sems.at[slot]))
                if p % 2 == 1:
                    computes.append(u_compute(p, bufs[p], slot, i))
                else:
                    computes.append(l_compute(p, bufs[p], slot, i))
            copies.append(subs)
            phase_of.append(p)
            g += 1

    n_steps = g
    for j in range(min(_PREF, n_steps)):
        for c in copies[j]:
            c.start()
    pending = []
    for g0 in range(0, n_steps, 2):
        pair = [g0] if g0 + 1 >= n_steps else [g0, g0 + 1]
        for g in pair:
            if g + _PREF < n_steps:
                for c in copies[g + _PREF]:
                    c.start()
        if pending and phase_of[g0] != phase_of[g0 - 1]:
            for pend_epi, pend_val in pending:
                pend_epi(pend_val)
            pending = []
        for g in pair:
            for c in copies[g]:
                c.wait()
        vals = [computes[g][0]() for g in pair]
        for pend_epi, pend_val in pending:
            pend_epi(pend_val)
        pending = [(computes[g][1], v) for g, v in zip(pair, vals)]
    for pend_epi, pend_val in pending:
        pend_epi(pend_val)


def _mega(latents, fcb2, Ub0, W0, b0, Ub1, W1, b1, Ub2, W2, b2,
          fc_W, U0, L0, U1, L1, U2, L2):
    f32 = jnp.float32
    vmem = pl.BlockSpec(memory_space=pltpu.MemorySpace.VMEM)
    hbm = pl.BlockSpec(memory_space=pltpu.MemorySpace.HBM)
    return pl.pallas_call(
        _mega_body,
        in_specs=[vmem] * 11 + [hbm] * 7,
        out_specs=vmem,
        out_shape=jax.ShapeDtypeStruct((_B, _NODES[2], _COUT[2]), f32),
        scratch_shapes=[
            pltpu.VMEM((_NBUF, _TM, 512), f32),
            pltpu.VMEM((_NBUF, _TM, 1024), f32),
            pltpu.VMEM((_NBUF, _TM, 2048), f32),
            pltpu.VMEM((_NBUF, _TM, 4096), f32),
            pltpu.VMEM((_B, _N0 * _CHANNELS[0]), f32),
            pltpu.VMEM((_NODES[0], _B * _CIN[0]), f32),
            pltpu.VMEM((_NODES[0], _B * _COUT[0]), f32),
            pltpu.VMEM((_NODES[1], _B * _COUT[0]), f32),
            pltpu.VMEM((_NODES[1], _B * _COUT[1]), f32),
            pltpu.VMEM((_NODES[2], _B * _COUT[1]), f32),
            pltpu.SemaphoreType.DMA((_NBUF,)),
        ],
    )(latents, fcb2, Ub0, W0, b0.reshape(1, -1), Ub1, W1, b1.reshape(1, -1),
      Ub2, W2, b2.reshape(1, -1), fc_W, U0, L0, U1, L1, U2, L2)


def kernel(latents, fc_W, fc_b, U0, Ub0, L0, W0, b0, U1, Ub1, L1, W1, b1, U2, Ub2, L2, W2, b2):
    fcb2 = fc_b.reshape(_N0, _CHANNELS[0])
    return _mega(latents, fcb2, Ub0, W0, b0, Ub1, W1, b1, Ub2, W2, b2,
                 fc_W, U0, L0, U1, L1, U2, L2)


# speedup vs baseline: 1.1088x; 1.1088x over previous
import jax
import jax.numpy as jnp
from jax.experimental import pallas as pl
from jax.experimental.pallas import tpu as pltpu

_B = 4
_LATENT = 128
_N0 = 512
_CHANNELS = [32, 16, 8]
_NODES = [1024, 2048, 4096]
_CIN = [32, 32, 16]
_COUT = [32, 16, 8]
_TM = 256
_TFC = 2048
_NBUF = 6
_PREF = 4
_SUB_ROWS = 32

_DOT_KW = dict(preferred_element_type=jnp.float32,
               precision=jax.lax.Precision.DEFAULT)


def _split(x_f32):
    x_hi = x_f32.astype(jnp.bfloat16).astype(jnp.float32)
    return x_hi, x_f32 - x_hi


def _block_diag(w, c_in, c_out):
    wt = jnp.tile(w, (_B, _B))
    rows = jax.lax.broadcasted_iota(jnp.int32, wt.shape, 0) // c_in
    cols = jax.lax.broadcasted_iota(jnp.int32, wt.shape, 1) // c_out
    return jnp.where(rows == cols, wt, 0.0)


def _mega_body(lat_ref, fcb_ref, ub0_ref, w0_ref, b0_ref, ub1_ref, w1_ref,
               b1_ref, ub2_ref, w2_ref, b2_ref,
               fcw_hbm, u0_hbm, l0_hbm, u1_hbm, l1_hbm, u2_hbm, l2_hbm,
               out_ref, buf_b, buf_c, buf_d, buf_e,
               fc_out, y0, x1, y1, x2, y2, sems):
    chain = (None, fc_out, y0, x1, y1, x2, y2, out_ref)
    hbm_refs = (fcw_hbm, u0_hbm, l0_hbm, u1_hbm, l1_hbm, u2_hbm, l2_hbm)
    bufs = (buf_d, buf_b, buf_c, buf_c, buf_d, buf_d, buf_e)
    n_tiles = (_N0 * _CHANNELS[0] // _TFC,) + tuple(
        _NODES[(p - 1) // 2] // _TM for p in range(1, 7))
    ub_refs = (ub0_ref, ub1_ref, ub2_ref)
    w_refs = (w0_ref, w1_ref, w2_ref)
    b_refs = (b0_ref, b1_ref, b2_ref)

    split_cache = {}
    const_cache = {}

    def get_cat(p):
        if p not in split_cache:
            if p == 1:
                f = fc_out[...].reshape(_B, _N0, _CHANNELS[0])
                x = f.transpose(1, 0, 2).reshape(_N0, _B * _CHANNELS[0])
                x = x + jnp.tile(fcb_ref[...], (1, _B))
            else:
                x = chain[p][...]
            x_hi, x_lo = _split(x)
            split_cache[p] = jnp.concatenate([x_hi, x_lo], axis=1)
        return split_cache[p]

    def get_lat_cat():
        if 'lat' not in split_cache:
            lat_hi, lat_lo = _split(lat_ref[...])
            split_cache['lat'] = jnp.concatenate([lat_hi, lat_lo], axis=0)
        return split_cache['lat']

    def get_consts(lvl):
        if lvl not in const_cache:
            const_cache[lvl] = (
                _block_diag(w_refs[lvl][...], _CIN[lvl], _COUT[lvl]),
                jnp.tile(b_refs[lvl][...], (1, _B)))
        return const_cache[lvl]

    def fc_compute(buf, slot, i):
        def main():
            w = buf[slot, 0:_LATENT, :]
            return jnp.dot(get_lat_cat(), w, **_DOT_KW)

        def epi(ff):
            f = ff[:_B, :] + ff[_B:, :]
            fc_out[:, i * _TFC:(i + 1) * _TFC] = f
        return main, epi

    def u_compute(p, buf, slot, i):
        lvl = (p - 1) // 2

        def main():
            return jnp.dot(buf[slot], get_cat(p), **_DOT_KW)

        def epi(yy):
            n = yy.shape[1] // 2
            y = yy[:, :n] + yy[:, n:]
            ub = ub_refs[lvl][i * _TM:(i + 1) * _TM, :]
            chain[p + 1][i * _TM:(i + 1) * _TM, :] = y + jnp.tile(ub, (1, _B))
        return main, epi

    def l_compute(p, buf, slot, i):
        lvl = (p - 1) // 2

        def main():
            return jnp.dot(buf[slot], get_cat(p), **_DOT_KW)

        def epi(zz):
            n = zz.shape[1] // 2
            z = zz[:, :n] + zz[:, n:]
            wbd, bt = get_consts(lvl)
            h = jnp.dot(z, wbd, **_DOT_KW)
            chain[p + 1][i * _TM:(i + 1) * _TM, :] = jnp.maximum(h + bt, 0.0)
        return main, epi

    copies = []
    computes = []
    phase_of = []
    g = 0
    for p in range(7):
        for i in range(n_tiles[p]):
            slot = g % _NBUF
            subs = []
            if p == 0:
                for r in range(0, _LATENT, _SUB_ROWS):
                    subs.append(pltpu.make_async_copy(
                        hbm_refs[0].at[r:r + _SUB_ROWS,
                                       i * _TFC:(i + 1) * _TFC],
                        bufs[0].at[slot, r:r + _SUB_ROWS, :],
                        sems.at[slot]))
                computes.append(fc_compute(bufs[0], slot, i))
            else:
                for r in range(0, _TM, _SUB_ROWS):
                    subs.append(pltpu.make_async_copy(
                        hbm_refs[p].at[i * _TM + r:i * _TM + r + _SUB_ROWS, :],
                        bufs[p].at[slot, r:r + _SUB_ROWS, :], sems.at[slot]))
                if p % 2 == 1:
                    computes.append(u_compute(p, bufs[p], slot, i))
                else:
                    computes.append(l_compute(p, bufs[p], slot, i))
            copies.append(subs)
            phase_of.append(p)
            g += 1

    n_steps = g
    for j in range(min(_PREF, n_steps)):
        for c in copies[j]:
            c.start()
    pending = []
    for g0 in range(0, n_steps, 2):
        pair = [g0] if g0 + 1 >= n_steps else [g0, g0 + 1]
        for g in pair:
            if g + _PREF < n_steps:
                for c in copies[g + _PREF]:
                    c.start()
        if pending and phase_of[g0] != phase_of[g0 - 1]:
            for pend_epi, pend_val in pending:
                pend_epi(pend_val)
            pending = []
        for g in pair:
            for c in copies[g]:
                c.wait()
        vals = [computes[g][0]() for g in pair]
        for pend_epi, pend_val in pending:
            pend_epi(pend_val)
        pending = [(computes[g][1], v) for g, v in zip(pair, vals)]
    for pend_epi, pend_val in pending:
        pend_epi(pend_val)


def _mega(latents, fcb2, Ub0, W0, b0, Ub1, W1, b1, Ub2, W2, b2,
          fc_W, U0, L0, U1, L1, U2, L2):
    f32 = jnp.float32
    vmem = pl.BlockSpec(memory_space=pltpu.MemorySpace.VMEM)
    hbm = pl.BlockSpec(memory_space=pltpu.MemorySpace.HBM)
    return pl.pallas_call(
        _mega_body,
        in_specs=[vmem] * 11 + [hbm] * 7,
        out_specs=vmem,
        out_shape=jax.ShapeDtypeStruct((_NODES[2], _B * _COUT[2]), f32),
        scratch_shapes=[
            pltpu.VMEM((_NBUF, _TM, 512), f32),
            pltpu.VMEM((_NBUF, _TM, 1024), f32),
            pltpu.VMEM((_NBUF, _TM, 2048), f32),
            pltpu.VMEM((_NBUF, _TM, 4096), f32),
            pltpu.VMEM((_B, _N0 * _CHANNELS[0]), f32),
            pltpu.VMEM((_NODES[0], _B * _CIN[0]), f32),
            pltpu.VMEM((_NODES[0], _B * _COUT[0]), f32),
            pltpu.VMEM((_NODES[1], _B * _COUT[0]), f32),
            pltpu.VMEM((_NODES[1], _B * _COUT[1]), f32),
            pltpu.VMEM((_NODES[2], _B * _COUT[1]), f32),
            pltpu.SemaphoreType.DMA((_NBUF,)),
        ],
    )(latents, fcb2, Ub0, W0, b0.reshape(1, -1), Ub1, W1, b1.reshape(1, -1),
      Ub2, W2, b2.reshape(1, -1), fc_W, U0, L0, U1, L1, U2, L2)


def kernel(latents, fc_W, fc_b, U0, Ub0, L0, W0, b0, U1, Ub1, L1, W1, b1, U2, Ub2, L2, W2, b2):
    fcb2 = fc_b.reshape(_N0, _CHANNELS[0])
    out = _mega(latents, fcb2, Ub0, W0, b0, Ub1, W1, b1, Ub2, W2, b2,
                fc_W, U0, L0, U1, L1, U2, L2)
    return out.reshape(_NODES[2], _B, _COUT[2]).transpose(1, 0, 2)
